# baseline (speedup 1.0000x reference)
import jax
import jax.numpy as jnp
from jax.experimental import pallas as pl
from jax.experimental.pallas import tpu as pltpu

_HIGHEST = jax.lax.Precision.HIGHEST


def _make_fused_kernel(ppd, C, D, HW):
    def _fused_kernel(x_ref, y_ref, p1_ref, p2_ref, o_ref):
        xv = x_ref[0]
        yv = y_ref[0]
        u = xv + yv
        pooled = jnp.dot(u, p1_ref[0:HW, :],
                         preferred_element_type=jnp.float32)
        acc = pooled[:, 0:1] * p1_ref[HW:HW + C, 0:D]
        for p in range(1, ppd):
            acc = acc + pooled[:, p:p + 1] * p1_ref[HW + p * C:
                                                    HW + (p + 1) * C, 0:D]
        b1f = p2_ref[2 * D + 2:2 * D + 3, 0:D]
        z = jnp.sum(acc, axis=0, keepdims=True) + b1f
        z = jnp.maximum(z, 0.0)
        zx = jnp.dot(z, p2_ref[0:D, :], precision=_HIGHEST,
                     preferred_element_type=jnp.float32) \
            + p2_ref[2 * D:2 * D + 1, :]
        zy = jnp.dot(z, p2_ref[D:2 * D, :], precision=_HIGHEST,
                     preferred_element_type=jnp.float32) \
            + p2_ref[2 * D + 1:2 * D + 2, :]
        m = jnp.maximum(zx, zy)
        ex = jnp.exp(zx - m)
        ey = jnp.exp(zy - m)
        wxr = ex / (ex + ey)
        rows = jax.lax.broadcasted_iota(jnp.int32, (C, C), 0)
        cols = jax.lax.broadcasted_iota(jnp.int32, (C, C), 1)
        wxc = jnp.sum(jnp.where(rows == cols, wxr, 0.0), axis=1,
                      keepdims=True)
        wyc = 1.0 - wxc
        o_ref[0] = (xv * wxc + yv * wyc).astype(o_ref.dtype)

    return _fused_kernel


def kernel(x, y, pmat, w1f, b1f, wx, bx, wy, by):
    B, C, H, W = x.shape
    HW = H * W
    D = w1f.shape[1]
    L = pmat.shape[1]
    ppd = w1f.shape[0] // C

    xf = x.reshape(B, C, HW)
    yf = y.reshape(B, C, HW)

    w1f3 = w1f.reshape(C, ppd, D).transpose(1, 0, 2).reshape(ppd * C, D)
    p1 = jnp.concatenate(
        [pmat, jnp.pad(w1f3, ((0, 0), (0, L - D)))], axis=0)
    p2 = jnp.concatenate(
        [wx, wy, bx.reshape(1, C), by.reshape(1, C),
         jnp.pad(b1f.reshape(1, D), ((0, 0), (0, C - D)))],
        axis=0)

    out = pl.pallas_call(
        _make_fused_kernel(ppd, C, D, HW),
        out_shape=jax.ShapeDtypeStruct((B, C, HW), x.dtype),
        grid=(B,),
        in_specs=[
            pl.BlockSpec((1, C, HW), lambda b: (b, 0, 0)),
            pl.BlockSpec((1, C, HW), lambda b: (b, 0, 0)),
            pl.BlockSpec((HW + ppd * C, L), lambda b: (0, 0)),
            pl.BlockSpec((2 * D + 3, C), lambda b: (0, 0)),
        ],
        out_specs=pl.BlockSpec((1, C, HW), lambda b: (b, 0, 0)),
        compiler_params=pltpu.CompilerParams(
            dimension_semantics=("parallel",),
            vmem_limit_bytes=48 << 20),
    )(xf, yf, p1, p2)

    return out.reshape(B, C, H, W), None, y


# speedup vs baseline: 1.0061x; 1.0061x over previous
import jax
import jax.numpy as jnp
from jax.experimental import pallas as pl
from jax.experimental.pallas import tpu as pltpu

_HIGHEST = jax.lax.Precision.HIGHEST


def _make_fused_kernel(ppd, C, D, HW):
    def _fused_kernel(x_ref, y_ref, pmat_ref, w1f3_ref, p2_ref, o_ref):
        xv = x_ref[0]
        yv = y_ref[0]
        u = xv + yv
        pooled = jnp.dot(u, pmat_ref[...],
                         preferred_element_type=jnp.float32)
        acc = pooled[:, 0:1] * w1f3_ref[0]
        for p in range(1, ppd):
            acc = acc + pooled[:, p:p + 1] * w1f3_ref[p]
        b1f = p2_ref[2 * D + 2:2 * D + 3, 0:D]
        z = jnp.sum(acc, axis=0, keepdims=True) + b1f
        z = jnp.maximum(z, 0.0)
        zx = jnp.dot(z, p2_ref[0:D, :], precision=_HIGHEST,
                     preferred_element_type=jnp.float32) \
            + p2_ref[2 * D:2 * D + 1, :]
        zy = jnp.dot(z, p2_ref[D:2 * D, :], precision=_HIGHEST,
                     preferred_element_type=jnp.float32) \
            + p2_ref[2 * D + 1:2 * D + 2, :]
        m = jnp.maximum(zx, zy)
        ex = jnp.exp(zx - m)
        ey = jnp.exp(zy - m)
        wxr = ex / (ex + ey)
        rows = jax.lax.broadcasted_iota(jnp.int32, (C, C), 0)
        cols = jax.lax.broadcasted_iota(jnp.int32, (C, C), 1)
        wxc = jnp.sum(jnp.where(rows == cols, wxr, 0.0), axis=1,
                      keepdims=True)
        wyc = 1.0 - wxc
        o_ref[0] = (xv * wxc + yv * wyc).astype(o_ref.dtype)

    return _fused_kernel


def kernel(x, y, pmat, w1f, b1f, wx, bx, wy, by):
    B, C, H, W = x.shape
    HW = H * W
    D = w1f.shape[1]
    L = pmat.shape[1]
    ppd = w1f.shape[0] // C

    xf = x.reshape(B, C, HW)
    yf = y.reshape(B, C, HW)

    w1f3 = w1f.reshape(C, ppd, D).transpose(1, 0, 2)
    p2 = jnp.concatenate(
        [wx, wy, bx.reshape(1, C), by.reshape(1, C),
         jnp.pad(b1f.reshape(1, D), ((0, 0), (0, C - D)))],
        axis=0)

    out = pl.pallas_call(
        _make_fused_kernel(ppd, C, D, HW),
        out_shape=jax.ShapeDtypeStruct((B, C, HW), x.dtype),
        grid=(B,),
        in_specs=[
            pl.BlockSpec((1, C, HW), lambda b: (b, 0, 0)),
            pl.BlockSpec((1, C, HW), lambda b: (b, 0, 0)),
            pl.BlockSpec((HW, L), lambda b: (0, 0)),
            pl.BlockSpec((ppd, C, D), lambda b: (0, 0, 0)),
            pl.BlockSpec((2 * D + 3, C), lambda b: (0, 0)),
        ],
        out_specs=pl.BlockSpec((1, C, HW), lambda b: (b, 0, 0)),
        compiler_params=pltpu.CompilerParams(
            dimension_semantics=("parallel",),
            vmem_limit_bytes=48 << 20),
    )(xf, yf, pmat, w1f3, p2)

    return out.reshape(B, C, H, W), None, y


# speedup vs baseline: 1.1254x; 1.1187x over previous
import jax
import jax.numpy as jnp
from jax.experimental import pallas as pl
from jax.experimental.pallas import tpu as pltpu

_HIGHEST = jax.lax.Precision.HIGHEST


def _make_fused_kernel(ppd, C, D, HW, nb):
    def _fused_kernel(x_ref, y_ref, pmat_ref, w1f3_ref, p2_ref, o_ref):
        u = (x_ref[...] + y_ref[...]).reshape(nb * C, HW)
        pooled = jnp.dot(u, pmat_ref[...],
                         preferred_element_type=jnp.float32)
        w3 = [jnp.tile(w1f3_ref[p], (nb, 1)) for p in range(ppd)]
        acc = pooled[:, 0:1] * w3[0]
        for p in range(1, ppd):
            acc = acc + pooled[:, p:p + 1] * w3[p]
        b1f = p2_ref[2 * D + 2:2 * D + 3, 0:D]
        z = jnp.sum(acc.reshape(nb, C, D), axis=1) + b1f
        z = jnp.maximum(z, 0.0)
        zx = jnp.dot(z, p2_ref[0:D, :], precision=_HIGHEST,
                     preferred_element_type=jnp.float32) \
            + p2_ref[2 * D:2 * D + 1, :]
        zy = jnp.dot(z, p2_ref[D:2 * D, :], precision=_HIGHEST,
                     preferred_element_type=jnp.float32) \
            + p2_ref[2 * D + 1:2 * D + 2, :]
        m = jnp.maximum(zx, zy)
        ex = jnp.exp(zx - m)
        ey = jnp.exp(zy - m)
        wxr = ex / (ex + ey)
        rows = jax.lax.broadcasted_iota(jnp.int32, (C, C), 0)
        cols = jax.lax.broadcasted_iota(jnp.int32, (C, C), 1)
        eye = rows == cols
        for i in range(nb):
            wxc = jnp.sum(jnp.where(eye, wxr[i:i + 1, :], 0.0), axis=1,
                          keepdims=True)
            wyc = 1.0 - wxc
            o_ref[i] = (x_ref[i] * wxc + y_ref[i] * wyc).astype(o_ref.dtype)

    return _fused_kernel


def kernel(x, y, pmat, w1f, b1f, wx, bx, wy, by):
    B, C, H, W = x.shape
    HW = H * W
    D = w1f.shape[1]
    L = pmat.shape[1]
    ppd = w1f.shape[0] // C
    nb = 4 if B % 4 == 0 else 1

    xf = x.reshape(B, C, HW)
    yf = y.reshape(B, C, HW)

    w1f3 = w1f.reshape(C, ppd, D).transpose(1, 0, 2)
    p2 = jnp.concatenate(
        [wx, wy, bx.reshape(1, C), by.reshape(1, C),
         jnp.pad(b1f.reshape(1, D), ((0, 0), (0, C - D)))],
        axis=0)

    out = pl.pallas_call(
        _make_fused_kernel(ppd, C, D, HW, nb),
        out_shape=jax.ShapeDtypeStruct((B, C, HW), x.dtype),
        grid=(B // nb,),
        in_specs=[
            pl.BlockSpec((nb, C, HW), lambda b: (b, 0, 0)),
            pl.BlockSpec((nb, C, HW), lambda b: (b, 0, 0)),
            pl.BlockSpec((HW, L), lambda b: (0, 0)),
            pl.BlockSpec((ppd, C, D), lambda b: (0, 0, 0)),
            pl.BlockSpec((2 * D + 3, C), lambda b: (0, 0)),
        ],
        out_specs=pl.BlockSpec((nb, C, HW), lambda b: (b, 0, 0)),
        compiler_params=pltpu.CompilerParams(
            dimension_semantics=("parallel",),
            vmem_limit_bytes=48 << 20),
    )(xf, yf, pmat, w1f3, p2)

    return out.reshape(B, C, H, W), None, y
